# baseline (speedup 1.0000x reference)
import functools

import jax
import jax.numpy as jnp
from jax.experimental import pallas as pl
from jax.experimental.pallas import tpu as pltpu

S, D, H, DFF, E = 2048, 1024, 16, 4096, 8
DH = D // H
BM = 256
NB = S * 2 // BM + E
PAD = NB * BM
BQ = 512


def _ln_qkv_body(x_ref, g_ref, b_ref, wq_ref, wk_ref, wv_ref,
                 q_ref, k_ref, v_ref):
    x = x_ref[...]
    m = jnp.mean(x, axis=-1, keepdims=True)
    v = jnp.mean(jnp.square(x - m), axis=-1, keepdims=True)
    h = (x - m) * jax.lax.rsqrt(v + 1e-5) * g_ref[...] + b_ref[...]
    q_ref[...] = jnp.dot(h, wq_ref[...], preferred_element_type=jnp.float32)
    k_ref[...] = jnp.dot(h, wk_ref[...], preferred_element_type=jnp.float32)
    v_ref[...] = jnp.dot(h, wv_ref[...], preferred_element_type=jnp.float32)


def _ln_qkv(x, g, b, wq, wk, wv):
    n = S // BM
    full = pl.BlockSpec((D, D), lambda i: (0, 0))
    row = pl.BlockSpec((BM, D), lambda i: (i, 0))
    vec = pl.BlockSpec((1, D), lambda i: (0, 0))
    return pl.pallas_call(
        _ln_qkv_body,
        grid=(n,),
        in_specs=[row, vec, vec, full, full, full],
        out_specs=[row, row, row],
        out_shape=[jax.ShapeDtypeStruct((S, D), jnp.float32)] * 3,
    )(x, g, b, wq, wk, wv)


def _attn_body(q_ref, k_ref, v_ref, o_ref):
    q = q_ref[0]
    k = k_ref[0]
    s = jax.lax.dot_general(q, k, (((1,), (1,)), ((), ())),
                            preferred_element_type=jnp.float32)
    s = s * (1.0 / (DH ** 0.5))
    s = s - jnp.max(s, axis=-1, keepdims=True)
    p = jnp.exp(s)
    p = p / jnp.sum(p, axis=-1, keepdims=True)
    o_ref[0] = jnp.dot(p, v_ref[0], preferred_element_type=jnp.float32)


def _attention(qh, kh, vh):
    return pl.pallas_call(
        _attn_body,
        grid=(H, S // BQ),
        in_specs=[
            pl.BlockSpec((1, BQ, DH), lambda h, i: (h, i, 0)),
            pl.BlockSpec((1, S, DH), lambda h, i: (h, 0, 0)),
            pl.BlockSpec((1, S, DH), lambda h, i: (h, 0, 0)),
        ],
        out_specs=pl.BlockSpec((1, BQ, DH), lambda h, i: (h, i, 0)),
        out_shape=jax.ShapeDtypeStruct((H, S, DH), jnp.float32),
    )(qh, kh, vh)


def _post_body(x_ref, o_ref, wo_ref, wg_ref, bg_ref, x2_ref, wd_ref):
    x2 = x_ref[...] + jnp.dot(o_ref[...], wo_ref[...],
                              preferred_element_type=jnp.float32)
    x2_ref[...] = x2
    logits = jnp.dot(x2, wg_ref[...], preferred_element_type=jnp.float32)
    logits = logits + bg_ref[...]
    col = jax.lax.broadcasted_iota(jnp.int32, logits.shape, 1)
    logits = jnp.where(col < E, logits, -1e30)
    i1 = jnp.argmax(logits, axis=-1)[:, None]
    m1 = jnp.max(logits, axis=-1, keepdims=True)
    l2 = jnp.where(col == i1, -1e30, logits)
    i2 = jnp.argmax(l2, axis=-1)[:, None]
    m2 = jnp.max(l2, axis=-1, keepdims=True)
    p1 = 1.0 / (1.0 + jnp.exp(m2 - m1))
    wd_ref[...] = jnp.where(col == i1, p1,
                            jnp.where(col == i2, 1.0 - p1, 0.0))


def _post_attn(x, o, wo, wg_pad, bg_pad):
    n = S // BM
    row = pl.BlockSpec((BM, D), lambda i: (i, 0))
    return pl.pallas_call(
        _post_body,
        grid=(n,),
        in_specs=[
            row, row,
            pl.BlockSpec((D, D), lambda i: (0, 0)),
            pl.BlockSpec((D, 128), lambda i: (0, 0)),
            pl.BlockSpec((1, 128), lambda i: (0, 0)),
        ],
        out_specs=[row, pl.BlockSpec((BM, 128), lambda i: (i, 0))],
        out_shape=[jax.ShapeDtypeStruct((S, D), jnp.float32),
                   jax.ShapeDtypeStruct((S, 128), jnp.float32)],
    )(x, o, wo, wg_pad, bg_pad)


BD = 2048
NJ = DFF // BD


def _moe_body(be_ref, xg_ref, w1_ref, b1_ref, w2_ref, b2_ref, y_ref, acc_ref):
    del be_ref
    j = pl.program_id(1)
    h = jnp.dot(xg_ref[...], w1_ref[0], preferred_element_type=jnp.float32)
    h = jax.nn.gelu(h + b1_ref[0])
    part = jnp.dot(h, w2_ref[0], preferred_element_type=jnp.float32)

    @pl.when(j == 0)
    def _():
        acc_ref[...] = part

    @pl.when(j > 0)
    def _():
        acc_ref[...] += part

    @pl.when(j == NJ - 1)
    def _():
        y_ref[...] = acc_ref[...] + b2_ref[0]


def _moe_gemm(xg, w1, b1, w2, b2, block_expert):
    grid_spec = pltpu.PrefetchScalarGridSpec(
        num_scalar_prefetch=1,
        grid=(NB, NJ),
        in_specs=[
            pl.BlockSpec((BM, D), lambda b, j, be: (b, 0)),
            pl.BlockSpec((1, D, BD), lambda b, j, be: (be[b], 0, j)),
            pl.BlockSpec((1, 1, BD), lambda b, j, be: (be[b], 0, j)),
            pl.BlockSpec((1, BD, D), lambda b, j, be: (be[b], j, 0)),
            pl.BlockSpec((1, 1, D), lambda b, j, be: (be[b], 0, 0)),
        ],
        out_specs=pl.BlockSpec((BM, D), lambda b, j, be: (b, 0)),
        scratch_shapes=[pltpu.VMEM((BM, D), jnp.float32)],
    )
    return pl.pallas_call(
        _moe_body,
        grid_spec=grid_spec,
        out_shape=jax.ShapeDtypeStruct((PAD, D), jnp.float32),
    )(block_expert, xg, w1, b1, w2, b2)


def _combine_body(x2_ref, y1_ref, y2_ref, w1_ref, w2_ref, o_ref):
    o_ref[...] = (x2_ref[...]
                  + w1_ref[:, 0:1] * y1_ref[...]
                  + w2_ref[:, 0:1] * y2_ref[...])


def _combine(x2, y1, y2, w1b, w2b):
    n = S // BM
    row = pl.BlockSpec((BM, D), lambda i: (i, 0))
    wrow = pl.BlockSpec((BM, 128), lambda i: (i, 0))
    return pl.pallas_call(
        _combine_body,
        grid=(n,),
        in_specs=[row, row, row, wrow, wrow],
        out_specs=row,
        out_shape=jax.ShapeDtypeStruct((S, D), jnp.float32),
    )(x2, y1, y2, w1b, w2b)


def _routing(wdense):
    ind = wdense > 0.0
    indi = ind.astype(jnp.int32)
    counts = jnp.sum(indi, axis=0)
    padded = ((counts + BM - 1) // BM) * BM
    pstart = jnp.concatenate([jnp.zeros((1,), jnp.int32),
                              jnp.cumsum(padded)[:-1].astype(jnp.int32)])
    rank = jnp.cumsum(indi, axis=0) - indi
    dst_te = pstart[None, :] + rank
    flat_dst = jnp.where(ind, dst_te, PAD).reshape(-1)
    tok = jnp.broadcast_to(jnp.arange(S, dtype=jnp.int32)[:, None],
                           (S, E)).reshape(-1)
    gidx = jnp.zeros((PAD + 1,), jnp.int32).at[flat_dst].set(
        tok, mode='drop')[:PAD]
    block_expert = (jnp.searchsorted(
        pstart, jnp.arange(NB, dtype=jnp.int32) * BM,
        side='right') - 1).astype(jnp.int32)
    ar = jnp.arange(S)
    e1 = jnp.argmax(ind, axis=1)
    e2 = E - 1 - jnp.argmax(ind[:, ::-1], axis=1)
    dst1 = dst_te[ar, e1]
    dst2 = dst_te[ar, e2]
    w1 = wdense[ar, e1]
    w2 = wdense[ar, e2]
    return gidx, block_expert, dst1, dst2, w1, w2


def kernel(x, Wq, Wk, Wv, Wo, g_attn, b_attn, W_task, b_task,
           W_gate, b_gate, W1, b1, W2, b2):
    del W_task, b_task
    x0 = x[0]
    g = g_attn[None, :]
    bb = b_attn[None, :]
    wg_pad = jnp.zeros((D, 128), jnp.float32).at[:, :E].set(W_gate)
    bg_pad = jnp.zeros((1, 128), jnp.float32).at[0, :E].set(b_gate)

    q, k, v = _ln_qkv(x0, g, bb, Wq, Wk, Wv)
    qh = q.reshape(S, H, DH).transpose(1, 0, 2)
    kh = k.reshape(S, H, DH).transpose(1, 0, 2)
    vh = v.reshape(S, H, DH).transpose(1, 0, 2)
    oh = _attention(qh, kh, vh)
    o = oh.transpose(1, 0, 2).reshape(S, D)
    x2, wd = _post_attn(x0, o, Wo, wg_pad, bg_pad)

    gidx, block_expert, dst1, dst2, w1, w2 = _routing(wd[:, :E])

    xg = jnp.take(x2, gidx, axis=0)
    ypad = _moe_gemm(xg, W1, b1[:, None, :], W2, b2[:, None, :], block_expert)
    y1 = jnp.take(ypad, dst1, axis=0)
    y2 = jnp.take(ypad, dst2, axis=0)
    w1b = jnp.broadcast_to(w1[:, None], (S, 128))
    w2b = jnp.broadcast_to(w2[:, None], (S, 128))
    out = _combine(x2, y1, y2, w1b, w2b)
    return out[None]


# speedup vs baseline: 1.1425x; 1.1425x over previous
import functools

import jax
import jax.numpy as jnp
from jax.experimental import pallas as pl
from jax.experimental.pallas import tpu as pltpu

S, D, H, DFF, E = 2048, 1024, 16, 4096, 8
DH = D // H
BM = 256
NB = S * 2 // BM + E
PAD = NB * BM
BQ = 512


def _ln_qkv_body(x_ref, g_ref, b_ref, wq_ref, wk_ref, wv_ref,
                 q_ref, k_ref, v_ref):
    x = x_ref[...]
    m = jnp.mean(x, axis=-1, keepdims=True)
    v = jnp.mean(jnp.square(x - m), axis=-1, keepdims=True)
    h = (x - m) * jax.lax.rsqrt(v + 1e-5) * g_ref[...] + b_ref[...]
    q_ref[...] = jnp.dot(h, wq_ref[...], preferred_element_type=jnp.float32)
    k_ref[...] = jnp.dot(h, wk_ref[...], preferred_element_type=jnp.float32)
    v_ref[...] = jnp.dot(h, wv_ref[...], preferred_element_type=jnp.float32)


def _ln_qkv(x, g, b, wq, wk, wv):
    n = S // BM
    full = pl.BlockSpec((D, D), lambda i: (0, 0))
    row = pl.BlockSpec((BM, D), lambda i: (i, 0))
    vec = pl.BlockSpec((1, D), lambda i: (0, 0))
    return pl.pallas_call(
        _ln_qkv_body,
        grid=(n,),
        in_specs=[row, vec, vec, full, full, full],
        out_specs=[row, row, row],
        out_shape=[jax.ShapeDtypeStruct((S, D), jnp.float32)] * 3,
    )(x, g, b, wq, wk, wv)


def _attn_body(q_ref, k_ref, v_ref, o_ref):
    q = q_ref[...]
    k = k_ref[...]
    v = v_ref[...]
    outs = []
    for h in range(H):
        sl = slice(h * DH, (h + 1) * DH)
        s = jax.lax.dot_general(q[:, sl], k[:, sl], (((1,), (1,)), ((), ())),
                                preferred_element_type=jnp.float32)
        s = s * (1.0 / (DH ** 0.5))
        s = s - jnp.max(s, axis=-1, keepdims=True)
        p = jnp.exp(s)
        p = p / jnp.sum(p, axis=-1, keepdims=True)
        outs.append(jnp.dot(p, v[:, sl], preferred_element_type=jnp.float32))
    o_ref[...] = jnp.concatenate(outs, axis=1)


def _attention(q, k, v):
    return pl.pallas_call(
        _attn_body,
        grid=(S // BQ,),
        in_specs=[
            pl.BlockSpec((BQ, D), lambda i: (i, 0)),
            pl.BlockSpec((S, D), lambda i: (0, 0)),
            pl.BlockSpec((S, D), lambda i: (0, 0)),
        ],
        out_specs=pl.BlockSpec((BQ, D), lambda i: (i, 0)),
        out_shape=jax.ShapeDtypeStruct((S, D), jnp.float32),
    )(q, k, v)


def _post_body(x_ref, o_ref, wo_ref, wg_ref, bg_ref, x2_ref, wd_ref):
    x2 = x_ref[...] + jnp.dot(o_ref[...], wo_ref[...],
                              preferred_element_type=jnp.float32)
    x2_ref[...] = x2
    logits = jnp.dot(x2, wg_ref[...], preferred_element_type=jnp.float32)
    logits = logits + bg_ref[...]
    col = jax.lax.broadcasted_iota(jnp.int32, logits.shape, 1)
    logits = jnp.where(col < E, logits, -1e30)
    i1 = jnp.argmax(logits, axis=-1)[:, None]
    m1 = jnp.max(logits, axis=-1, keepdims=True)
    l2 = jnp.where(col == i1, -1e30, logits)
    i2 = jnp.argmax(l2, axis=-1)[:, None]
    m2 = jnp.max(l2, axis=-1, keepdims=True)
    p1 = 1.0 / (1.0 + jnp.exp(m2 - m1))
    wd_ref[...] = jnp.where(col == i1, p1,
                            jnp.where(col == i2, 1.0 - p1, 0.0))


def _post_attn(x, o, wo, wg_pad, bg_pad):
    n = S // BM
    row = pl.BlockSpec((BM, D), lambda i: (i, 0))
    return pl.pallas_call(
        _post_body,
        grid=(n,),
        in_specs=[
            row, row,
            pl.BlockSpec((D, D), lambda i: (0, 0)),
            pl.BlockSpec((D, 128), lambda i: (0, 0)),
            pl.BlockSpec((1, 128), lambda i: (0, 0)),
        ],
        out_specs=[row, pl.BlockSpec((BM, 128), lambda i: (i, 0))],
        out_shape=[jax.ShapeDtypeStruct((S, D), jnp.float32),
                   jax.ShapeDtypeStruct((S, 128), jnp.float32)],
    )(x, o, wo, wg_pad, bg_pad)


BD = 2048
NJ = DFF // BD


def _moe_body(be_ref, xg_ref, w1_ref, b1_ref, w2_ref, b2_ref, y_ref, acc_ref):
    del be_ref
    j = pl.program_id(1)
    xb = xg_ref[...].astype(jnp.bfloat16)
    w1b = w1_ref[0].astype(jnp.bfloat16)
    h = jnp.dot(xb, w1b, preferred_element_type=jnp.float32)
    h = jax.nn.gelu(h + b1_ref[0]).astype(jnp.bfloat16)
    w2b = w2_ref[0].astype(jnp.bfloat16)
    part = jnp.dot(h, w2b, preferred_element_type=jnp.float32)

    @pl.when(j == 0)
    def _():
        acc_ref[...] = part

    @pl.when(j > 0)
    def _():
        acc_ref[...] += part

    @pl.when(j == NJ - 1)
    def _():
        y_ref[...] = acc_ref[...] + b2_ref[0]


def _moe_gemm(xg, w1, b1, w2, b2, block_expert):
    grid_spec = pltpu.PrefetchScalarGridSpec(
        num_scalar_prefetch=1,
        grid=(NB, NJ),
        in_specs=[
            pl.BlockSpec((BM, D), lambda b, j, be: (b, 0)),
            pl.BlockSpec((1, D, BD), lambda b, j, be: (be[b], 0, j)),
            pl.BlockSpec((1, 1, BD), lambda b, j, be: (be[b], 0, j)),
            pl.BlockSpec((1, BD, D), lambda b, j, be: (be[b], j, 0)),
            pl.BlockSpec((1, 1, D), lambda b, j, be: (be[b], 0, 0)),
        ],
        out_specs=pl.BlockSpec((BM, D), lambda b, j, be: (b, 0)),
        scratch_shapes=[pltpu.VMEM((BM, D), jnp.float32)],
    )
    return pl.pallas_call(
        _moe_body,
        grid_spec=grid_spec,
        out_shape=jax.ShapeDtypeStruct((PAD, D), jnp.float32),
    )(block_expert, xg, w1, b1, w2, b2)


def _combine_body(x2_ref, y1_ref, y2_ref, w1_ref, w2_ref, o_ref):
    o_ref[...] = (x2_ref[...]
                  + w1_ref[:, 0:1] * y1_ref[...]
                  + w2_ref[:, 0:1] * y2_ref[...])


def _combine(x2, y1, y2, w1b, w2b):
    n = S // BM
    row = pl.BlockSpec((BM, D), lambda i: (i, 0))
    wrow = pl.BlockSpec((BM, 128), lambda i: (i, 0))
    return pl.pallas_call(
        _combine_body,
        grid=(n,),
        in_specs=[row, row, row, wrow, wrow],
        out_specs=row,
        out_shape=jax.ShapeDtypeStruct((S, D), jnp.float32),
    )(x2, y1, y2, w1b, w2b)


def _routing(wdense):
    ind = wdense > 0.0
    indi = ind.astype(jnp.int32)
    counts = jnp.sum(indi, axis=0)
    padded = ((counts + BM - 1) // BM) * BM
    pstart = jnp.concatenate([jnp.zeros((1,), jnp.int32),
                              jnp.cumsum(padded)[:-1].astype(jnp.int32)])
    rank = jnp.cumsum(indi, axis=0) - indi
    dst_te = pstart[None, :] + rank
    flat_dst = jnp.where(ind, dst_te, PAD).reshape(-1)
    tok = jnp.broadcast_to(jnp.arange(S, dtype=jnp.int32)[:, None],
                           (S, E)).reshape(-1)
    gidx = jnp.zeros((PAD + 1,), jnp.int32).at[flat_dst].set(
        tok, mode='drop')[:PAD]
    block_expert = (jnp.searchsorted(
        pstart, jnp.arange(NB, dtype=jnp.int32) * BM,
        side='right') - 1).astype(jnp.int32)
    ar = jnp.arange(S)
    e1 = jnp.argmax(ind, axis=1)
    e2 = E - 1 - jnp.argmax(ind[:, ::-1], axis=1)
    dst1 = dst_te[ar, e1]
    dst2 = dst_te[ar, e2]
    w1 = wdense[ar, e1]
    w2 = wdense[ar, e2]
    return gidx, block_expert, dst1, dst2, w1, w2


def kernel(x, Wq, Wk, Wv, Wo, g_attn, b_attn, W_task, b_task,
           W_gate, b_gate, W1, b1, W2, b2):
    del W_task, b_task
    x0 = x[0]
    g = g_attn[None, :]
    bb = b_attn[None, :]
    wg_pad = jnp.zeros((D, 128), jnp.float32).at[:, :E].set(W_gate)
    bg_pad = jnp.zeros((1, 128), jnp.float32).at[0, :E].set(b_gate)

    q, k, v = _ln_qkv(x0, g, bb, Wq, Wk, Wv)
    o = _attention(q, k, v)
    x2, wd = _post_attn(x0, o, Wo, wg_pad, bg_pad)

    gidx, block_expert, dst1, dst2, w1, w2 = _routing(wd[:, :E])

    xg = jnp.take(x2, gidx, axis=0)
    ypad = _moe_gemm(xg, W1, b1[:, None, :], W2, b2[:, None, :], block_expert)
    y1 = jnp.take(ypad, dst1, axis=0)
    y2 = jnp.take(ypad, dst2, axis=0)
    w1b = jnp.broadcast_to(w1[:, None], (S, 128))
    w2b = jnp.broadcast_to(w2[:, None], (S, 128))
    out = _combine(x2, y1, y2, w1b, w2b)
    return out[None]
